# Initial kernel scaffold; baseline (speedup 1.0000x reference)
import jax
import jax.numpy as jnp
from jax.experimental import pallas as pl


def kernel(x, edge_index, edge_attr, W1, b1, W2, b2, Wc1, bc1, Wc2, bc2):
    raise NotImplementedError("write your pallas kernel here")


# speedup vs baseline: 10.3357x; 10.3357x over previous
import functools

import jax
import jax.numpy as jnp
from jax import lax
from jax.experimental import pallas as pl
from jax.experimental.pallas import tpu as pltpu
from jax.experimental.pallas import tpu_sc as plsc

N = 10000
E = 320000
D_NODE = 128
H = 64
L = 16
NC = 2
NS = 16
NW = NC * NS
EPW = E // NW
IB = 80
NBLK = EPW // IB
NP = 10240
RPS = NP // NS
RZ = 128
NCHUNK = RPS // RZ

_f32 = jnp.float32


def _mesh():
    return plsc.VectorSubcoreMesh(core_axis_name="c", subcore_axis_name="s")


_SC_PARAMS = pltpu.CompilerParams(use_tc_tiling_on_sc=False)


def _fill_rows(ref, nrows, ncols, val):
    @pl.loop(0, nrows)
    def _(r):
        @pl.loop(0, ncols // L)
        def _(c):
            ref[pl.ds(r, 1), pl.ds(c * L, L)] = jnp.full((1, L), val, ref.dtype)


def _sc_degree(dst3):
    @functools.partial(
        pl.kernel,
        out_type=jax.ShapeDtypeStruct((NC, NP, L), _f32),
        mesh=_mesh(),
        compiler_params=_SC_PARAMS,
        scratch_types=[
            pltpu.VMEM((NBLK, IB), jnp.int32),
            pltpu.VMEM((IB, L), _f32),
            pltpu.VMEM((RPS, L), _f32),
            pltpu.VMEM_SHARED((NP, L), _f32),
        ],
    )
    def k(dst_hbm, out_hbm, idx_v, ones_v, z_v, acc_sh):
        cid = lax.axis_index("c")
        sid = lax.axis_index("s")
        wid = sid * NC + cid
        _fill_rows(ones_v, IB, L, 1.0)
        _fill_rows(z_v, RPS, L, 0.0)
        pltpu.sync_copy(z_v, acc_sh.at[pl.ds(sid * RPS, RPS)])
        plsc.subcore_barrier()
        pltpu.sync_copy(dst_hbm.at[wid], idx_v)

        @pl.loop(0, NBLK)
        def _(j):
            pltpu.sync_copy(ones_v, acc_sh.at[idx_v.at[j]], add=True)

        plsc.subcore_barrier()
        pltpu.sync_copy(
            acc_sh.at[pl.ds(sid * RPS, RPS)],
            out_hbm.at[cid, pl.ds(sid * RPS, RPS)],
        )

    return k(dst3)


def _sc_segsum(table, src3, dst3):
    @functools.partial(
        pl.kernel,
        out_type=jax.ShapeDtypeStruct((NC, NP, H), _f32),
        mesh=_mesh(),
        compiler_params=_SC_PARAMS,
        scratch_types=[
            pltpu.VMEM((NBLK, IB), jnp.int32),
            pltpu.VMEM((NBLK, IB), jnp.int32),
            pltpu.VMEM((IB, H), _f32),
            pltpu.VMEM((RZ, H), _f32),
            pltpu.VMEM_SHARED((NP, H), _f32),
            pltpu.SemaphoreType.DMA,
        ],
    )
    def k(table_hbm, src_hbm, dst_hbm, out_hbm, sidx_v, didx_v, rows_v, z_v,
          acc_sh, sem):
        cid = lax.axis_index("c")
        sid = lax.axis_index("s")
        wid = sid * NC + cid
        _fill_rows(z_v, RZ, H, 0.0)

        @pl.loop(0, NCHUNK)
        def _(kk):
            pltpu.sync_copy(z_v, acc_sh.at[pl.ds(sid * RPS + kk * RZ, RZ)])

        plsc.subcore_barrier()
        pltpu.sync_copy(src_hbm.at[wid], sidx_v)
        pltpu.sync_copy(dst_hbm.at[wid], didx_v)

        @pl.loop(0, NBLK)
        def _(j):
            pltpu.async_copy(table_hbm.at[sidx_v.at[j]], rows_v, sem).wait()
            pltpu.sync_copy(rows_v, acc_sh.at[didx_v.at[j]], add=True)

        plsc.subcore_barrier()

        @pl.loop(0, NCHUNK)
        def _(kk):
            r0 = sid * RPS + kk * RZ
            pltpu.sync_copy(acc_sh.at[pl.ds(r0, RZ)],
                            out_hbm.at[cid, pl.ds(r0, RZ)])

    return k(table, src3, dst3)


def _sc_edge_gather(a_tab, b_tab, src3, dst3):
    @functools.partial(
        pl.kernel,
        out_type=jax.ShapeDtypeStruct((E, H), _f32),
        mesh=_mesh(),
        compiler_params=_SC_PARAMS,
        scratch_types=[
            pltpu.VMEM((NBLK, IB), jnp.int32),
            pltpu.VMEM((NBLK, IB), jnp.int32),
            pltpu.VMEM((IB, H), _f32),
            pltpu.VMEM((IB, H), _f32),
            pltpu.VMEM((IB,), jnp.int32),
            pltpu.VMEM_SHARED((NS * IB, H), _f32),
            pltpu.SemaphoreType.DMA,
            pltpu.SemaphoreType.DMA,
        ],
    )
    def k(a_hbm, b_hbm, src_hbm, dst_hbm, out_hbm, sidx_v, didx_v, a_v, b_v,
          myidx_v, stage_sh, sem_a, sem_b):
        cid = lax.axis_index("c")
        sid = lax.axis_index("s")
        wid = sid * NC + cid

        @pl.loop(0, IB // L)
        def _(kk):
            myidx_v[pl.ds(kk * L, L)] = (lax.iota(jnp.int32, L) + kk * L
                                         + sid * IB)

        pltpu.sync_copy(src_hbm.at[wid], sidx_v)
        pltpu.sync_copy(dst_hbm.at[wid], didx_v)
        base = wid * EPW

        @pl.loop(0, NBLK)
        def _(j):
            ca = pltpu.async_copy(a_hbm.at[sidx_v.at[j]], a_v, sem_a)
            cb = pltpu.async_copy(b_hbm.at[didx_v.at[j]], b_v, sem_b)
            ca.wait()
            cb.wait()
            pltpu.sync_copy(a_v, stage_sh.at[pl.ds(sid * IB, IB)])
            pltpu.sync_copy(b_v, stage_sh.at[myidx_v], add=True)
            pltpu.sync_copy(stage_sh.at[pl.ds(sid * IB, IB)],
                            out_hbm.at[pl.ds(base + j * IB, IB)])

    return k(a_tab, b_tab, src3, dst3)


def _tc_prescale(degp, x, W1):
    def body(degp_ref, x_ref, w_ref, h1p_ref, dinv_ref):
        deg = 1.0 + degp_ref[0, 0:N, 0:1] + degp_ref[1, 0:N, 0:1]
        dinv = lax.rsqrt(deg)
        g = jnp.dot(x_ref[...], w_ref[...], preferred_element_type=_f32)
        h1p_ref[...] = dinv * g
        dinv_ref[...] = dinv

    return pl.pallas_call(
        body,
        out_shape=[jax.ShapeDtypeStruct((N, H), _f32),
                   jax.ShapeDtypeStruct((N, 1), _f32)],
    )(degp, x, W1)


def _tc_mid(p, h1p, dinv, W2, b1):
    def body(p_ref, h1p_ref, dinv_ref, w_ref, b_ref, h2p_ref):
        dv = dinv_ref[...]
        h1 = jnp.maximum(dv * (p_ref[0, 0:N, :] + p_ref[1, 0:N, :] + h1p_ref[...]) + b_ref[...],
                         0.0)
        g2 = jnp.dot(h1, w_ref[...], preferred_element_type=_f32)
        h2p_ref[...] = dv * g2

    return pl.pallas_call(
        body,
        out_shape=jax.ShapeDtypeStruct((N, H), _f32),
    )(p, h1p, dinv, W2, b1.reshape(1, H))


def _tc_post(q, h2p, dinv, b2, Wcs, Wcd):
    def body(q_ref, h2p_ref, dinv_ref, b_ref, ws_ref, wd_ref, as_ref, ad_ref):
        h2 = dinv_ref[...] * (q_ref[0, 0:N, :] + q_ref[1, 0:N, :] + h2p_ref[...]) + b_ref[...]
        as_ref[...] = jnp.dot(h2, ws_ref[...], preferred_element_type=_f32)
        ad_ref[...] = jnp.dot(h2, wd_ref[...], preferred_element_type=_f32)

    return pl.pallas_call(
        body,
        out_shape=[jax.ShapeDtypeStruct((N, H), _f32),
                   jax.ShapeDtypeStruct((N, H), _f32)],
    )(q, h2p, dinv, b2.reshape(1, H), Wcs, Wcd)


_BE = 3200


def _tc_classifier(S, edge_attr, Wce, bc1, Wc2, bc2):
    def body(s_ref, ea_ref, wce_ref, b1_ref, w2_ref, b2_ref, out_ref):
        z = s_ref[...] + jnp.dot(ea_ref[...], wce_ref[...],
                                 preferred_element_type=_f32) + b1_ref[...]
        z = jnp.maximum(z, 0.0)
        out_ref[...] = jnp.dot(z, w2_ref[...],
                               preferred_element_type=_f32) + b2_ref[...]

    return pl.pallas_call(
        body,
        grid=(E // _BE,),
        in_specs=[
            pl.BlockSpec((_BE, H), lambda i: (i, 0)),
            pl.BlockSpec((_BE, 16), lambda i: (i, 0)),
            pl.BlockSpec((16, H), lambda i: (0, 0)),
            pl.BlockSpec((1, H), lambda i: (0, 0)),
            pl.BlockSpec((H, 8), lambda i: (0, 0)),
            pl.BlockSpec((1, 8), lambda i: (0, 0)),
        ],
        out_specs=pl.BlockSpec((_BE, 8), lambda i: (i, 0)),
        out_shape=jax.ShapeDtypeStruct((E, 8), _f32),
    )(S, edge_attr, Wce, bc1, Wc2, bc2)


def kernel(x, edge_index, edge_attr, W1, b1, W2, b2, Wc1, bc1, Wc2, bc2):
    src3 = edge_index[0].reshape(NW, NBLK, IB)
    dst3 = edge_index[1].reshape(NW, NBLK, IB)
    degp = _sc_degree(dst3)
    h1p, dinv = _tc_prescale(degp, x, W1)
    p1 = _sc_segsum(h1p, src3, dst3)
    h2p = _tc_mid(p1, h1p, dinv, W2, b1)
    p2 = _sc_segsum(h2p, src3, dst3)
    As, Ad = _tc_post(p2, h2p, dinv, b2, Wc1[:H], Wc1[H:2 * H])
    S = _sc_edge_gather(As, Ad, src3, dst3)
    return _tc_classifier(S, edge_attr, Wc1[2 * H:], bc1.reshape(1, H), Wc2,
                          bc2.reshape(1, 8))


# speedup vs baseline: 14.0501x; 1.3594x over previous
import functools

import jax
import jax.numpy as jnp
from jax import lax
from jax.experimental import pallas as pl
from jax.experimental.pallas import tpu as pltpu
from jax.experimental.pallas import tpu_sc as plsc

N = 10000
E = 320000
D_NODE = 128
H = 64
L = 16
NC = 2
NS = 16
NW = NC * NS
IB = 128
BPW = 79
EPW = BPW * IB
EP = NW * EPW
NP = 10240
RPS = NP // NS
RZ = 128
NCHUNK = RPS // RZ

_f32 = jnp.float32


def _mesh():
    return plsc.VectorSubcoreMesh(core_axis_name="c", subcore_axis_name="s")


_SC_PARAMS = pltpu.CompilerParams(use_tc_tiling_on_sc=False)


def _fill_rows(ref, nrows, ncols, val):
    @pl.loop(0, nrows)
    def _(r):
        @pl.loop(0, ncols // L)
        def _(c):
            ref[pl.ds(r, 1), pl.ds(c * L, L)] = jnp.full((1, L), val, ref.dtype)


def _sc_degree(dst3):
    @functools.partial(
        pl.kernel,
        out_type=jax.ShapeDtypeStruct((NC, NP, L), _f32),
        mesh=_mesh(),
        compiler_params=_SC_PARAMS,
        scratch_types=[
            pltpu.VMEM((BPW, IB), jnp.int32),
            pltpu.VMEM((IB, L), _f32),
            pltpu.VMEM((RPS, L), _f32),
            pltpu.VMEM_SHARED((NP, L), _f32),
        ],
    )
    def k(dst_hbm, out_hbm, idx_v, ones_v, z_v, acc_sh):
        cid = lax.axis_index("c")
        sid = lax.axis_index("s")
        wid = sid * NC + cid
        _fill_rows(ones_v, IB, L, 1.0)
        _fill_rows(z_v, RPS, L, 0.0)
        pltpu.sync_copy(z_v, acc_sh.at[pl.ds(sid * RPS, RPS)])
        plsc.subcore_barrier()
        pltpu.sync_copy(dst_hbm.at[wid], idx_v)

        @pl.loop(0, BPW)
        def _(j):
            pltpu.sync_copy(ones_v, acc_sh.at[idx_v.at[j]], add=True)

        plsc.subcore_barrier()
        pltpu.sync_copy(
            acc_sh.at[pl.ds(sid * RPS, RPS)],
            out_hbm.at[cid, pl.ds(sid * RPS, RPS)],
        )

    return k(dst3)


def _sc_segsum(table, src3, dst3):
    @functools.partial(
        pl.kernel,
        out_type=jax.ShapeDtypeStruct((NC, NP, H), _f32),
        mesh=_mesh(),
        compiler_params=_SC_PARAMS,
        scratch_types=[
            pltpu.VMEM((BPW, IB), jnp.int32),
            pltpu.VMEM((BPW, IB), jnp.int32),
            pltpu.VMEM((IB, H), _f32),
            pltpu.VMEM((IB, H), _f32),
            pltpu.VMEM((RZ, H), _f32),
            pltpu.VMEM_SHARED((NP, H), _f32),
            pltpu.SemaphoreType.DMA,
            pltpu.SemaphoreType.DMA,
        ],
    )
    def k(table_hbm, src_hbm, dst_hbm, out_hbm, sidx_v, didx_v, r0_v, r1_v,
          z_v, acc_sh, sem0, sem1):
        cid = lax.axis_index("c")
        sid = lax.axis_index("s")
        wid = sid * NC + cid
        _fill_rows(z_v, RZ, H, 0.0)

        @pl.loop(0, NCHUNK)
        def _(kk):
            pltpu.sync_copy(z_v, acc_sh.at[pl.ds(sid * RPS + kk * RZ, RZ)])

        plsc.subcore_barrier()
        pltpu.sync_copy(src_hbm.at[wid], sidx_v)
        pltpu.sync_copy(dst_hbm.at[wid], didx_v)

        def g_start(j, buf, sem):
            pltpu.make_async_copy(table_hbm.at[sidx_v.at[j]], buf, sem).start()

        def g_wait(j, buf, sem):
            pltpu.make_async_copy(table_hbm.at[sidx_v.at[j]], buf, sem).wait()

        g_start(0, r0_v, sem0)

        @pl.loop(0, (BPW - 1) // 2)
        def _(jj):
            j = 2 * jj
            g_start(j + 1, r1_v, sem1)
            g_wait(j, r0_v, sem0)
            pltpu.sync_copy(r0_v, acc_sh.at[didx_v.at[j]], add=True)
            g_start(j + 2, r0_v, sem0)
            g_wait(j + 1, r1_v, sem1)
            pltpu.sync_copy(r1_v, acc_sh.at[didx_v.at[j + 1]], add=True)

        g_wait(BPW - 1, r0_v, sem0)
        pltpu.sync_copy(r0_v, acc_sh.at[didx_v.at[BPW - 1]], add=True)
        plsc.subcore_barrier()

        @pl.loop(0, NCHUNK)
        def _(kk):
            r0 = sid * RPS + kk * RZ
            pltpu.sync_copy(acc_sh.at[pl.ds(r0, RZ)],
                            out_hbm.at[cid, pl.ds(r0, RZ)])

    return k(table, src3, dst3)


def _sc_edge_gather(a_tab, b_tab, src3, dst3):
    @functools.partial(
        pl.kernel,
        out_type=jax.ShapeDtypeStruct((EP, H), _f32),
        mesh=_mesh(),
        compiler_params=_SC_PARAMS,
        scratch_types=[
            pltpu.VMEM((BPW, IB), jnp.int32),
            pltpu.VMEM((BPW, IB), jnp.int32),
            pltpu.VMEM((IB, H), _f32),
            pltpu.VMEM((IB, H), _f32),
            pltpu.VMEM((IB, H), _f32),
            pltpu.VMEM((IB, H), _f32),
            pltpu.VMEM((IB,), jnp.int32),
            pltpu.VMEM_SHARED((NS * IB, H), _f32),
            pltpu.VMEM_SHARED((NS * IB, H), _f32),
            pltpu.SemaphoreType.DMA,
            pltpu.SemaphoreType.DMA,
            pltpu.SemaphoreType.DMA,
            pltpu.SemaphoreType.DMA,
            pltpu.SemaphoreType.DMA,
            pltpu.SemaphoreType.DMA,
        ],
    )
    def k(a_hbm, b_hbm, src_hbm, dst_hbm, out_hbm, sidx_v, didx_v,
          a0_v, a1_v, b0_v, b1_v, myidx_v, st0_sh, st1_sh,
          sa0, sa1, sb0, sb1, so0, so1):
        cid = lax.axis_index("c")
        sid = lax.axis_index("s")
        wid = sid * NC + cid

        @pl.loop(0, IB // L)
        def _(kk):
            myidx_v[pl.ds(kk * L, L)] = (lax.iota(jnp.int32, L) + kk * L
                                         + sid * IB)

        pltpu.sync_copy(src_hbm.at[wid], sidx_v)
        pltpu.sync_copy(dst_hbm.at[wid], didx_v)
        base = wid * EPW
        slab = pl.ds(sid * IB, IB)

        def ga_start(j, buf, sem):
            pltpu.make_async_copy(a_hbm.at[sidx_v.at[j]], buf, sem).start()

        def ga_wait(j, buf, sem):
            pltpu.make_async_copy(a_hbm.at[sidx_v.at[j]], buf, sem).wait()

        def gb_start(j, buf, sem):
            pltpu.make_async_copy(b_hbm.at[didx_v.at[j]], buf, sem).start()

        def gb_wait(j, buf, sem):
            pltpu.make_async_copy(b_hbm.at[didx_v.at[j]], buf, sem).wait()

        def o_start(j, st, sem):
            pltpu.make_async_copy(st.at[slab],
                                  out_hbm.at[pl.ds(base + j * IB, IB)],
                                  sem).start()

        def o_wait(j, st, sem):
            pltpu.make_async_copy(st.at[slab],
                                  out_hbm.at[pl.ds(base + j * IB, IB)],
                                  sem).wait()

        ga_start(0, a0_v, sa0)
        gb_start(0, b0_v, sb0)

        @pl.loop(0, (BPW - 1) // 2)
        def _(jj):
            j = 2 * jj
            ga_start(j + 1, a1_v, sa1)
            gb_start(j + 1, b1_v, sb1)
            ga_wait(j, a0_v, sa0)
            gb_wait(j, b0_v, sb0)

            @pl.when(jj > 0)
            def _():
                o_wait(j - 2, st0_sh, so0)

            pltpu.sync_copy(a0_v, st0_sh.at[slab])
            pltpu.sync_copy(b0_v, st0_sh.at[myidx_v], add=True)
            o_start(j, st0_sh, so0)

            ga_start(j + 2, a0_v, sa0)
            gb_start(j + 2, b0_v, sb0)
            ga_wait(j + 1, a1_v, sa1)
            gb_wait(j + 1, b1_v, sb1)

            @pl.when(jj > 0)
            def _():
                o_wait(j - 1, st1_sh, so1)

            pltpu.sync_copy(a1_v, st1_sh.at[slab])
            pltpu.sync_copy(b1_v, st1_sh.at[myidx_v], add=True)
            o_start(j + 1, st1_sh, so1)

        ga_wait(BPW - 1, a0_v, sa0)
        gb_wait(BPW - 1, b0_v, sb0)
        o_wait(BPW - 3, st0_sh, so0)
        pltpu.sync_copy(a0_v, st0_sh.at[slab])
        pltpu.sync_copy(b0_v, st0_sh.at[myidx_v], add=True)
        o_start(BPW - 1, st0_sh, so0)
        o_wait(BPW - 2, st1_sh, so1)
        o_wait(BPW - 1, st0_sh, so0)

    return k(a_tab, b_tab, src3, dst3)


def _tc_matmul(x, W1):
    def body(x_ref, w_ref, g_ref):
        g_ref[...] = jnp.dot(x_ref[...], w_ref[...],
                             preferred_element_type=_f32)

    return pl.pallas_call(
        body,
        out_shape=jax.ShapeDtypeStruct((N, H), _f32),
    )(x, W1)


def _tc_prescale(degp, g1):
    def body(degp_ref, g_ref, h1p_ref, dinv_ref):
        deg = 1.0 + degp_ref[0, 0:N, 0:1] + degp_ref[1, 0:N, 0:1]
        dinv = lax.rsqrt(deg)
        h1p_ref[0:N, :] = dinv * g_ref[...]
        h1p_ref[N:NP, :] = jnp.zeros((NP - N, H), _f32)
        dinv_ref[...] = dinv

    return pl.pallas_call(
        body,
        out_shape=[jax.ShapeDtypeStruct((NP, H), _f32),
                   jax.ShapeDtypeStruct((N, 1), _f32)],
    )(degp, g1)


def _tc_mid(p, h1p, dinv, W2, b1):
    def body(p_ref, h1p_ref, dinv_ref, w_ref, b_ref, h2p_ref):
        dv = dinv_ref[...]
        h1 = jnp.maximum(
            dv * (p_ref[0, 0:N, :] + p_ref[1, 0:N, :] + h1p_ref[0:N, :])
            + b_ref[...], 0.0)
        g2 = jnp.dot(h1, w_ref[...], preferred_element_type=_f32)
        h2p_ref[0:N, :] = dv * g2
        h2p_ref[N:NP, :] = jnp.zeros((NP - N, H), _f32)

    return pl.pallas_call(
        body,
        out_shape=jax.ShapeDtypeStruct((NP, H), _f32),
    )(p, h1p, dinv, W2, b1.reshape(1, H))


def _tc_post(q, h2p, dinv, b2, Wcs, Wcd):
    def body(q_ref, h2p_ref, dinv_ref, b_ref, ws_ref, wd_ref, as_ref, ad_ref):
        h2 = (dinv_ref[...]
              * (q_ref[0, 0:N, :] + q_ref[1, 0:N, :] + h2p_ref[0:N, :])
              + b_ref[...])
        as_ref[0:N, :] = jnp.dot(h2, ws_ref[...], preferred_element_type=_f32)
        as_ref[N:NP, :] = jnp.zeros((NP - N, H), _f32)
        ad_ref[0:N, :] = jnp.dot(h2, wd_ref[...], preferred_element_type=_f32)
        ad_ref[N:NP, :] = jnp.zeros((NP - N, H), _f32)

    return pl.pallas_call(
        body,
        out_shape=[jax.ShapeDtypeStruct((NP, H), _f32),
                   jax.ShapeDtypeStruct((NP, H), _f32)],
    )(q, h2p, dinv, b2.reshape(1, H), Wcs, Wcd)


_BE = 3200


def _tc_classifier(S, edge_attr, Wce, bc1, Wc2, bc2):
    def body(s_ref, ea_ref, wce_ref, b1_ref, w2_ref, b2_ref, out_ref):
        z = s_ref[...] + jnp.dot(ea_ref[...], wce_ref[...],
                                 preferred_element_type=_f32) + b1_ref[...]
        z = jnp.maximum(z, 0.0)
        out_ref[...] = jnp.dot(z, w2_ref[...],
                               preferred_element_type=_f32) + b2_ref[...]

    return pl.pallas_call(
        body,
        grid=(E // _BE,),
        in_specs=[
            pl.BlockSpec((_BE, H), lambda i: (i, 0)),
            pl.BlockSpec((_BE, 16), lambda i: (i, 0)),
            pl.BlockSpec((16, H), lambda i: (0, 0)),
            pl.BlockSpec((1, H), lambda i: (0, 0)),
            pl.BlockSpec((H, 8), lambda i: (0, 0)),
            pl.BlockSpec((1, 8), lambda i: (0, 0)),
        ],
        out_specs=pl.BlockSpec((_BE, 8), lambda i: (i, 0)),
        out_shape=jax.ShapeDtypeStruct((E, 8), _f32),
    )(S, edge_attr, Wce, bc1, Wc2, bc2)


def kernel(x, edge_index, edge_attr, W1, b1, W2, b2, Wc1, bc1, Wc2, bc2):
    pad = (N + jnp.arange(EP - E, dtype=jnp.int32) % (NP - N))
    src3 = jnp.concatenate([edge_index[0], pad]).reshape(NW, BPW, IB)
    dst3 = jnp.concatenate([edge_index[1], pad]).reshape(NW, BPW, IB)
    degp = _sc_degree(dst3)
    g1 = _tc_matmul(x, W1)
    h1p, dinv = _tc_prescale(degp, g1)
    p1 = _sc_segsum(h1p, src3, dst3)
    h2p = _tc_mid(p1, h1p, dinv, W2, b1)
    p2 = _sc_segsum(h2p, src3, dst3)
    As, Ad = _tc_post(p2, h2p, dinv, b2, Wc1[:H], Wc1[H:2 * H])
    S = _sc_edge_gather(As, Ad, src3, dst3)
    return _tc_classifier(S, edge_attr, Wc1[2 * H:], bc1.reshape(1, H), Wc2,
                          bc2.reshape(1, 8))
